# baseline (speedup 1.0000x reference)
import numpy as np

import jax
import jax.numpy as jnp
from jax import lax
from jax.experimental import pallas as pl
from jax.experimental.pallas import tpu as pltpu
from jax.experimental.pallas import tpu_sc as plsc

B = 1024
N = 50000
D = 256
N0 = 16
N1 = 16
L1 = 256
L2 = 128
R = B * N1
NC, NS = 2, 16
NW = NC * NS
RW = R // NW
BW = B // NW
G = 8
CH = G * N0

_PREC = jax.lax.Precision.HIGHEST
_LANES = D // 16


def _emit_segment_sum(feats, idx_v, out_hbm, out_row0, ndst, bufs, sems,
                      obuf, osem):
    nch = (ndst * N0) // CH
    niter = nch // 2

    def gather(ch, buf, sem):
        return pltpu.async_copy(
            feats.at[idx_v.at[pl.ds(ch * CH, CH)]], buf, sem)

    def reduce_buf(buf, obase):
        ngrp = D // 32
        himask = jnp.int32(-65536)

        def unpk(w):
            lo = lax.bitcast_convert_type(
                lax.shift_left(w, jnp.full((16,), 16, jnp.int32)),
                jnp.float32)
            hi = lax.bitcast_convert_type(jnp.bitwise_and(w, himask),
                                          jnp.float32)
            return lo, hi

        def rne(v):
            return v + 0x7FFF + jnp.bitwise_and(
                lax.shift_right_logical(v, jnp.full((16,), 16, jnp.int32)),
                1)

        def red(m, _):
            row = m * N0
            acc = [list(unpk(buf[row, pl.ds(c * 16, 16)]))
                   for c in range(ngrp)]
            for j in range(1, N0):
                for c in range(ngrp):
                    ea, eb = unpk(buf[row + j, pl.ds(c * 16, 16)])
                    acc[c][0] += ea
                    acc[c][1] += eb
            for c in range(ngrp):
                wl = rne(lax.bitcast_convert_type(acc[c][0], jnp.int32))
                wh = rne(lax.bitcast_convert_type(acc[c][1], jnp.int32))
                obuf[obase + m, pl.ds(c * 16, 16)] = jnp.bitwise_or(
                    jnp.bitwise_and(
                        lax.shift_right_logical(
                            wl, jnp.full((16,), 16, jnp.int32)), 0xFFFF),
                    jnp.bitwise_and(wh, jnp.int32(-65536)))
            return 0

        lax.fori_loop(0, G, red, 0, unroll=False)

    gather(0, bufs[0], sems[0])
    gather(1, bufs[1], sems[1])

    def body(i, _):
        @pl.when(i > 0)
        def _():
            pltpu.make_async_copy(
                out_hbm.at[pl.ds(0, 2 * G), :], obuf, osem).wait()

        pltpu.make_async_copy(feats.at[pl.ds(0, CH), :], bufs[0],
                              sems[0]).wait()
        reduce_buf(bufs[0], 0)

        @pl.when(i < niter - 1)
        def _():
            gather(2 * i + 2, bufs[0], sems[0])

        pltpu.make_async_copy(feats.at[pl.ds(0, CH), :], bufs[1],
                              sems[1]).wait()
        reduce_buf(bufs[1], G)

        @pl.when(i < niter - 1)
        def _():
            gather(2 * i + 3, bufs[1], sems[1])

        pltpu.async_copy(obuf, out_hbm.at[pl.ds(out_row0 + i * 2 * G, 2 * G), :],
                         osem)
        return 0

    lax.fori_loop(0, niter, body, 0, unroll=False)
    pltpu.make_async_copy(out_hbm.at[pl.ds(0, 2 * G), :], obuf, osem).wait()


def _sc_gather_kernel(feats, idx0w, n1w, idx10w, sum0_out, hp2_out,
                      sum10_out, idxa, idxb, idxc, buf0, buf1, obuf,
                      sem0, sem1, osem):
    wid = lax.axis_index("s") * NC + lax.axis_index("c")
    bufs = (buf0, buf1)
    sems = (sem0, sem1)

    pltpu.sync_copy(idx10w.at[wid], idxa)
    pltpu.sync_copy(n1w.at[wid], idxb)
    pltpu.sync_copy(idx0w.at[wid], idxc)

    _emit_segment_sum(feats, idxa, sum10_out, wid * RW, RW, bufs, sems,
                      obuf, osem)

    nch = RW // CH
    cps = [pltpu.async_copy(feats.at[idxb.at[pl.ds(k * CH, CH)]],
                            bufs[k % 2], sems[k % 2])
           for k in range(2)]
    for k in range(nch):
        cps[k % 2].wait()
        pltpu.sync_copy(bufs[k % 2],
                        hp2_out.at[pl.ds(wid * RW + k * CH, CH), :])
        if k + 2 < nch:
            cps[k % 2] = pltpu.async_copy(
                feats.at[idxb.at[pl.ds((k + 2) * CH, CH)]],
                bufs[k % 2], sems[k % 2])

    _emit_segment_sum(feats, idxc, sum0_out, wid * BW, BW, bufs, sems,
                      obuf, osem)


@jax.jit
def _sc_gather(feats, idx0w, n1w, idx10w):
    mesh = plsc.VectorSubcoreMesh(core_axis_name="c", subcore_axis_name="s",
                                  num_cores=NC, num_subcores=NS)
    return pl.kernel(
        _sc_gather_kernel,
        out_type=[
            jax.ShapeDtypeStruct((B, D // 2), jnp.int32),
            jax.ShapeDtypeStruct((R, D // 2), jnp.int32),
            jax.ShapeDtypeStruct((R, D // 2), jnp.int32),
        ],
        mesh=mesh,
        scratch_types=[
            pltpu.VMEM((RW * N0,), jnp.int32),
            pltpu.VMEM((RW,), jnp.int32),
            pltpu.VMEM((BW * N0,), jnp.int32),
            pltpu.VMEM((CH, D // 2), jnp.int32),
            pltpu.VMEM((CH, D // 2), jnp.int32),
            pltpu.VMEM((2 * G, D // 2), jnp.int32),
            pltpu.SemaphoreType.DMA,
            pltpu.SemaphoreType.DMA,
            pltpu.SemaphoreType.DMA,
        ],
    )(feats, idx0w, n1w, idx10w)


def _pack_body(f_ref, out_ref):
    w = lax.bitcast_convert_type(f_ref[...], jnp.int32)
    a, b = w[:, :D // 2], w[:, D // 2:]

    def rne(v):
        return v + 0x7FFF + jnp.bitwise_and(lax.shift_right_logical(v, 16), 1)

    lo = jnp.bitwise_and(lax.shift_right_logical(rne(a), 16), 0xFFFF)
    hi = jnp.bitwise_and(rne(b), jnp.int32(-65536))
    out_ref[...] = jnp.bitwise_or(lo, hi)


_PACK_BLK = 4000


@jax.jit
def _pack_table(feats):
    grid = (N + _PACK_BLK - 1) // _PACK_BLK
    return pl.pallas_call(
        _pack_body,
        grid=(grid,),
        in_specs=[pl.BlockSpec((_PACK_BLK, D), lambda i: (i, 0))],
        out_specs=pl.BlockSpec((_PACK_BLK, D // 2), lambda i: (i, 0)),
        out_shape=jax.ShapeDtypeStruct((N, D // 2), jnp.int32),
    )(feats)


_H2_BLK = 2048
_NBLK = R // _H2_BLK
_SBLK = _H2_BLK // N1


def _bf(v):
    return v.astype(jnp.bfloat16)


def _h2_body(hp2i_ref, s10_ref, x_ref, s0_ref, w0_ref, b0_ref, w1_ref,
             b1_ref, out_ref, ssum_acc):
    i = pl.program_id(0)
    def unpk2(w):
        lo = lax.bitcast_convert_type(lax.shift_left(w, 16), jnp.float32)
        hi = lax.bitcast_convert_type(jnp.bitwise_and(w, jnp.int32(-65536)),
                                      jnp.float32)
        return _bf(lo), _bf(hi)

    hlo, hhi = unpk2(hp2i_ref[...])
    slo, shi = unpk2(s10_ref[...])
    w0b16 = _bf(w0_ref[...])
    h = jnp.dot(hlo, w0b16[:D // 2], preferred_element_type=jnp.float32)
    h += jnp.dot(hhi, w0b16[D // 2:D], preferred_element_type=jnp.float32)
    h += jnp.dot(slo, w0b16[D:D + D // 2], preferred_element_type=jnp.float32)
    h += jnp.dot(shi, w0b16[D + D // 2:], preferred_element_type=jnp.float32)
    h += b0_ref[...][None, :]
    h = jnp.maximum(h, 0.0)
    ssum_acc[pl.ds(i * _SBLK, _SBLK), :] = jnp.sum(
        h.reshape(_SBLK, N1, L1), axis=1)

    @pl.when(i == _NBLK - 1)
    def _():
        zlo, zhi = unpk2(s0_ref[...])
        hp = jnp.dot(_bf(x_ref[...]), w0b16[:D],
                     preferred_element_type=jnp.float32)
        hp += jnp.dot(zlo, w0b16[D:D + D // 2],
                      preferred_element_type=jnp.float32)
        hp += jnp.dot(zhi, w0b16[D + D // 2:],
                      preferred_element_type=jnp.float32)
        hp += b0_ref[...][None, :]
        hp = jnp.maximum(hp, 0.0)
        w1b16 = _bf(w1_ref[...])
        o = jnp.dot(_bf(hp), w1b16[:L1], preferred_element_type=jnp.float32)
        o += jnp.dot(_bf(ssum_acc[...]), w1b16[L1:],
                     preferred_element_type=jnp.float32)
        o += b1_ref[...][None, :]
        out_ref[...] = jnp.maximum(o, 0.0)


@jax.jit
def _tc_forward(x, sum0, hp2i, sum10, W0, b0, W1, b1):
    zero = lambda i: (0, 0)
    return pl.pallas_call(
        _h2_body,
        grid=(_NBLK,),
        in_specs=[
            pl.BlockSpec((_H2_BLK, D // 2), lambda i: (i, 0)),
            pl.BlockSpec((_H2_BLK, D // 2), lambda i: (i, 0)),
            pl.BlockSpec((B, D), zero),
            pl.BlockSpec((B, D // 2), zero),
            pl.BlockSpec((2 * D, L1), zero),
            pl.BlockSpec((L1,), lambda i: (0,)),
            pl.BlockSpec((2 * L1, L2), zero),
            pl.BlockSpec((L2,), lambda i: (0,)),
        ],
        out_specs=pl.BlockSpec((B, L2), zero),
        out_shape=jax.ShapeDtypeStruct((B, L2), jnp.float32),
        scratch_shapes=[pltpu.VMEM((B, L1), jnp.float32)],
    )(hp2i, sum10, x, sum0, W0, b0, W1, b1)


def kernel(x, nodes, feats, neighs0, neighs1, neighs1_0, W0, b0, W1, b1):
    idx0w = neighs0.reshape(NW, BW * N0)
    n1w = neighs1.reshape(NW, RW)
    idx10w = neighs1_0.reshape(NW, RW * N0)
    fpacked = _pack_table(feats)
    sum0, hp2i, sum10 = _sc_gather(fpacked, idx0w, n1w, idx10w)
    W0_eff = jnp.concatenate([W0[:D], W0[D:] * (1.0 / N0)], axis=0)
    W1_eff = jnp.concatenate([W1[:L1], W1[L1:] * (1.0 / N1)], axis=0)
    return _tc_forward(x, sum0, hp2i, sum10, W0_eff, b0, W1_eff, b1)


# speedup vs baseline: 1.0492x; 1.0492x over previous
import numpy as np

import jax
import jax.numpy as jnp
from jax import lax
from jax.experimental import pallas as pl
from jax.experimental.pallas import tpu as pltpu
from jax.experimental.pallas import tpu_sc as plsc

B = 1024
N = 50000
D = 256
N0 = 16
N1 = 16
L1 = 256
L2 = 128
R = B * N1
NC, NS = 2, 16
NW = NC * NS
RW = R // NW
BW = B // NW
G = 16
CH = G * N0
IS = 128
NIS = CH // IS

_PREC = jax.lax.Precision.HIGHEST
_LANES = D // 16


def _emit_segment_sum(feats, idx_v, out_hbm, out_row0, ndst, bufs, sems,
                      obuf, osem):
    nch = (ndst * N0) // CH
    niter = nch // 2

    def gather(ch, buf, sem):
        for p in range(NIS):
            pltpu.async_copy(
                feats.at[idx_v.at[pl.ds(ch * CH + p * IS, IS)]],
                buf.at[pl.ds(p * IS, IS)], sem)

    def wait_buf(buf, sem):
        pltpu.make_async_copy(feats.at[pl.ds(0, CH), :], buf, sem).wait()

    def reduce_buf(buf, obase):
        ngrp = D // 32
        himask = jnp.int32(-65536)

        def unpk(w):
            lo = lax.bitcast_convert_type(
                lax.shift_left(w, jnp.full((16,), 16, jnp.int32)),
                jnp.float32)
            hi = lax.bitcast_convert_type(jnp.bitwise_and(w, himask),
                                          jnp.float32)
            return lo, hi

        def rne(v):
            return v + 0x7FFF + jnp.bitwise_and(
                lax.shift_right_logical(v, jnp.full((16,), 16, jnp.int32)),
                1)

        def red(m, _):
            row = m * N0
            acc = [list(unpk(buf[row, pl.ds(c * 16, 16)]))
                   for c in range(ngrp)]
            for j in range(1, N0):
                for c in range(ngrp):
                    ea, eb = unpk(buf[row + j, pl.ds(c * 16, 16)])
                    acc[c][0] += ea
                    acc[c][1] += eb
            for c in range(ngrp):
                wl = rne(lax.bitcast_convert_type(acc[c][0], jnp.int32))
                wh = rne(lax.bitcast_convert_type(acc[c][1], jnp.int32))
                obuf[obase + m, pl.ds(c * 16, 16)] = jnp.bitwise_or(
                    jnp.bitwise_and(
                        lax.shift_right_logical(
                            wl, jnp.full((16,), 16, jnp.int32)), 0xFFFF),
                    jnp.bitwise_and(wh, jnp.int32(-65536)))
            return 0

        lax.fori_loop(0, G, red, 0, unroll=False)

    gather(0, bufs[0], sems[0])
    gather(1, bufs[1], sems[1])

    def body(i, _):
        @pl.when(i > 0)
        def _():
            pltpu.make_async_copy(
                out_hbm.at[pl.ds(0, 2 * G), :], obuf, osem).wait()

        wait_buf(bufs[0], sems[0])
        reduce_buf(bufs[0], 0)

        @pl.when(i < niter - 1)
        def _():
            gather(2 * i + 2, bufs[0], sems[0])

        wait_buf(bufs[1], sems[1])
        reduce_buf(bufs[1], G)

        @pl.when(i < niter - 1)
        def _():
            gather(2 * i + 3, bufs[1], sems[1])

        pltpu.async_copy(obuf, out_hbm.at[pl.ds(out_row0 + i * 2 * G, 2 * G), :],
                         osem)
        return 0

    lax.fori_loop(0, niter, body, 0, unroll=False)
    pltpu.make_async_copy(out_hbm.at[pl.ds(0, 2 * G), :], obuf, osem).wait()


def _sc_gather_kernel(feats, idx0w, n1w, idx10w, sum0_out, hp2_out,
                      sum10_out, idxa, idxb, idxc, buf0, buf1, obuf,
                      sem0, sem1, osem):
    wid = lax.axis_index("s") * NC + lax.axis_index("c")
    bufs = (buf0, buf1)
    sems = (sem0, sem1)

    pltpu.sync_copy(idx10w.at[wid], idxa)
    pltpu.sync_copy(n1w.at[wid], idxb)
    pltpu.sync_copy(idx0w.at[wid], idxc)

    _emit_segment_sum(feats, idxa, sum10_out, wid * RW, RW, bufs, sems,
                      obuf, osem)

    nch = RW // CH
    for k in range(nch):
        for p in range(NIS):
            pltpu.async_copy(
                feats.at[idxb.at[pl.ds(k * CH + p * IS, IS)]],
                bufs[k % 2].at[pl.ds(p * IS, IS)], sems[k % 2])
    for k in range(nch):
        pltpu.make_async_copy(feats.at[pl.ds(0, CH), :], bufs[k % 2],
                              sems[k % 2]).wait()
        pltpu.sync_copy(bufs[k % 2],
                        hp2_out.at[pl.ds(wid * RW + k * CH, CH), :])

    _emit_segment_sum(feats, idxc, sum0_out, wid * BW, BW, bufs, sems,
                      obuf, osem)


@jax.jit
def _sc_gather(feats, idx0w, n1w, idx10w):
    mesh = plsc.VectorSubcoreMesh(core_axis_name="c", subcore_axis_name="s",
                                  num_cores=NC, num_subcores=NS)
    return pl.kernel(
        _sc_gather_kernel,
        out_type=[
            jax.ShapeDtypeStruct((B, D // 2), jnp.int32),
            jax.ShapeDtypeStruct((R, D // 2), jnp.int32),
            jax.ShapeDtypeStruct((R, D // 2), jnp.int32),
        ],
        mesh=mesh,
        scratch_types=[
            pltpu.VMEM((RW * N0,), jnp.int32),
            pltpu.VMEM((RW,), jnp.int32),
            pltpu.VMEM((BW * N0,), jnp.int32),
            pltpu.VMEM((CH, D // 2), jnp.int32),
            pltpu.VMEM((CH, D // 2), jnp.int32),
            pltpu.VMEM((2 * G, D // 2), jnp.int32),
            pltpu.SemaphoreType.DMA,
            pltpu.SemaphoreType.DMA,
            pltpu.SemaphoreType.DMA,
        ],
    )(feats, idx0w, n1w, idx10w)


def _pack_body(f_ref, out_ref):
    w = lax.bitcast_convert_type(f_ref[...], jnp.int32)
    a, b = w[:, :D // 2], w[:, D // 2:]

    def rne(v):
        return v + 0x7FFF + jnp.bitwise_and(lax.shift_right_logical(v, 16), 1)

    lo = jnp.bitwise_and(lax.shift_right_logical(rne(a), 16), 0xFFFF)
    hi = jnp.bitwise_and(rne(b), jnp.int32(-65536))
    out_ref[...] = jnp.bitwise_or(lo, hi)


_PACK_BLK = 4000


@jax.jit
def _pack_table(feats):
    grid = (N + _PACK_BLK - 1) // _PACK_BLK
    return pl.pallas_call(
        _pack_body,
        grid=(grid,),
        in_specs=[pl.BlockSpec((_PACK_BLK, D), lambda i: (i, 0))],
        out_specs=pl.BlockSpec((_PACK_BLK, D // 2), lambda i: (i, 0)),
        out_shape=jax.ShapeDtypeStruct((N, D // 2), jnp.int32),
    )(feats)


_H2_BLK = 4096
_NBLK = R // _H2_BLK
_SBLK = _H2_BLK // N1


def _bf(v):
    return v.astype(jnp.bfloat16)


def _h2_body(hp2i_ref, s10_ref, x_ref, s0_ref, w0_ref, b0_ref, w1_ref,
             b1_ref, out_ref, ssum_acc):
    i = pl.program_id(0)
    def unpk2(w):
        lo = lax.bitcast_convert_type(lax.shift_left(w, 16), jnp.float32)
        hi = lax.bitcast_convert_type(jnp.bitwise_and(w, jnp.int32(-65536)),
                                      jnp.float32)
        return _bf(lo), _bf(hi)

    hlo, hhi = unpk2(hp2i_ref[...])
    slo, shi = unpk2(s10_ref[...])
    w0b16 = _bf(w0_ref[...])
    h = jnp.dot(hlo, w0b16[:D // 2], preferred_element_type=jnp.float32)
    h += jnp.dot(hhi, w0b16[D // 2:D], preferred_element_type=jnp.float32)
    h += jnp.dot(slo, w0b16[D:D + D // 2], preferred_element_type=jnp.float32)
    h += jnp.dot(shi, w0b16[D + D // 2:], preferred_element_type=jnp.float32)
    h += b0_ref[...][None, :]
    h = jnp.maximum(h, 0.0)
    ssum_acc[pl.ds(i * _SBLK, _SBLK), :] = jnp.sum(
        h.reshape(_SBLK, N1, L1), axis=1)

    @pl.when(i == _NBLK - 1)
    def _():
        zlo, zhi = unpk2(s0_ref[...])
        hp = jnp.dot(_bf(x_ref[...]), w0b16[:D],
                     preferred_element_type=jnp.float32)
        hp += jnp.dot(zlo, w0b16[D:D + D // 2],
                      preferred_element_type=jnp.float32)
        hp += jnp.dot(zhi, w0b16[D + D // 2:],
                      preferred_element_type=jnp.float32)
        hp += b0_ref[...][None, :]
        hp = jnp.maximum(hp, 0.0)
        w1b16 = _bf(w1_ref[...])
        o = jnp.dot(_bf(hp), w1b16[:L1], preferred_element_type=jnp.float32)
        o += jnp.dot(_bf(ssum_acc[...]), w1b16[L1:],
                     preferred_element_type=jnp.float32)
        o += b1_ref[...][None, :]
        out_ref[...] = jnp.maximum(o, 0.0)


@jax.jit
def _tc_forward(x, sum0, hp2i, sum10, W0, b0, W1, b1):
    zero = lambda i: (0, 0)
    return pl.pallas_call(
        _h2_body,
        grid=(_NBLK,),
        in_specs=[
            pl.BlockSpec((_H2_BLK, D // 2), lambda i: (i, 0)),
            pl.BlockSpec((_H2_BLK, D // 2), lambda i: (i, 0)),
            pl.BlockSpec((B, D), zero),
            pl.BlockSpec((B, D // 2), zero),
            pl.BlockSpec((2 * D, L1), zero),
            pl.BlockSpec((L1,), lambda i: (0,)),
            pl.BlockSpec((2 * L1, L2), zero),
            pl.BlockSpec((L2,), lambda i: (0,)),
        ],
        out_specs=pl.BlockSpec((B, L2), zero),
        out_shape=jax.ShapeDtypeStruct((B, L2), jnp.float32),
        scratch_shapes=[pltpu.VMEM((B, L1), jnp.float32)],
    )(hp2i, sum10, x, sum0, W0, b0, W1, b1)


def kernel(x, nodes, feats, neighs0, neighs1, neighs1_0, W0, b0, W1, b1):
    idx0w = neighs0.reshape(NW, BW * N0)
    n1w = neighs1.reshape(NW, RW)
    idx10w = neighs1_0.reshape(NW, RW * N0)
    fpacked = _pack_table(feats)
    sum0, hp2i, sum10 = _sc_gather(fpacked, idx0w, n1w, idx10w)
    W0_eff = jnp.concatenate([W0[:D], W0[D:] * (1.0 / N0)], axis=0)
    W1_eff = jnp.concatenate([W1[:L1], W1[L1:] * (1.0 / N1)], axis=0)
    return _tc_forward(x, sum0, hp2i, sum10, W0_eff, b0, W1_eff, b1)
